# Initial kernel scaffold; baseline (speedup 1.0000x reference)
import jax
import jax.numpy as jnp
from jax.experimental import pallas as pl


def kernel(x, edge_index, Wl0, bl0, Wr0, Wl1, bl1, Wr1, Wl2, bl2, Wr2):
    raise NotImplementedError("write your pallas kernel here")


# speedup vs baseline: 4.4963x; 4.4963x over previous
import functools

import jax
import jax.numpy as jnp
from jax import lax
from jax.experimental import pallas as pl
from jax.experimental.pallas import tpu as pltpu
from jax.experimental.pallas import tpu_sc as plsc

NC = 2
NS = 16
NW = NC * NS
LANES = 16
CH = 128


def _dotT(a, b):
    return lax.dot_general(a, b, (((1,), (1,)), ((), ())),
                           preferred_element_type=jnp.float32)


def _tc_pre_body(x_ref, wl_ref, wr_ref, bl_ref, m_ref, r_ref):
    xb = x_ref[...]
    m_ref[...] = _dotT(xb, wl_ref[...])
    r_ref[...] = _dotT(xb, wr_ref[...]) + bl_ref[...]


def _tc_mid_body(p_ref, cnt_ref, r_ref, wl_ref, wr_ref, bl_ref, m_ref, rn_ref):
    cb = cnt_ref[...]
    cnt = cb[0, :, 0:1] + cb[1, :, 0:1]
    inv = 1.0 / jnp.maximum(cnt, 1.0)
    h = (p_ref[0] + p_ref[1]) * inv + r_ref[...]
    h = jnp.maximum(h, 0.0)
    m_ref[...] = _dotT(h, wl_ref[...])
    rn_ref[...] = _dotT(h, wr_ref[...]) + bl_ref[...]


def _tc_fin_body(p_ref, cnt_ref, r_ref, o_ref):
    cb = cnt_ref[...]
    cnt = cb[0, :, 0:1] + cb[1, :, 0:1]
    inv = 1.0 / jnp.maximum(cnt, 1.0)
    o_ref[...] = (p_ref[0] + p_ref[1]) * inv + r_ref[...]


def _make_sc_agg(n_nodes, np_rows, d, nchunk):
    mesh = plsc.VectorSubcoreMesh(core_axis_name="c", subcore_axis_name="s")
    rpt = np_rows // NS

    @functools.partial(
        pl.kernel,
        out_type=jax.ShapeDtypeStruct((NC, np_rows, d), jnp.float32),
        mesh=mesh,
        scratch_types=[
            pltpu.VMEM_SHARED((np_rows, d), jnp.float32),
            pltpu.VMEM((nchunk, CH), jnp.int32),
            pltpu.VMEM((nchunk, CH), jnp.int32),
            pltpu.VMEM((CH, d), jnp.float32),
            pltpu.SemaphoreType.DMA,
        ],
    )
    def sc_agg(m_hbm, src_hbm, dst_hbm, z_hbm, out_hbm,
               acc_sh, src_v, dst_v, rows_v, sem):
        c = lax.axis_index("c")
        s = lax.axis_index("s")
        wid = s * NC + c
        r0 = s * rpt
        pltpu.sync_copy(z_hbm.at[pl.ds(r0, rpt)], acc_sh.at[pl.ds(r0, rpt)])
        pltpu.sync_copy(src_hbm.at[wid], src_v)
        pltpu.sync_copy(dst_hbm.at[wid], dst_v)
        plsc.subcore_barrier()

        def step(j, carry):
            pltpu.async_copy(m_hbm.at[src_v.at[j]], rows_v, sem).wait()
            pltpu.sync_copy(rows_v, acc_sh.at[dst_v.at[j]], add=True)
            return carry

        lax.fori_loop(0, nchunk, step, 0, unroll=False)
        plsc.subcore_barrier()
        pltpu.sync_copy(acc_sh.at[pl.ds(r0, rpt)],
                        out_hbm.at[c, pl.ds(r0, rpt)])

    return sc_agg


def _make_sc_cnt(np_rows, d, nchunk):
    mesh = plsc.VectorSubcoreMesh(core_axis_name="c", subcore_axis_name="s")
    rpt = np_rows // NS

    @functools.partial(
        pl.kernel,
        out_type=jax.ShapeDtypeStruct((NC, np_rows, d), jnp.float32),
        mesh=mesh,
        scratch_types=[
            pltpu.VMEM_SHARED((np_rows, d), jnp.float32),
            pltpu.VMEM((nchunk, CH), jnp.int32),
            pltpu.VMEM((CH, d), jnp.float32),
        ],
    )
    def sc_cnt(dst_hbm, ones_hbm, z_hbm, out_hbm, cnt_sh, dst_v, ones_v):
        c = lax.axis_index("c")
        s = lax.axis_index("s")
        wid = s * NC + c
        r0 = s * rpt
        pltpu.sync_copy(z_hbm.at[pl.ds(r0, rpt)], cnt_sh.at[pl.ds(r0, rpt)])
        pltpu.sync_copy(dst_hbm.at[wid], dst_v)
        pltpu.sync_copy(ones_hbm, ones_v)
        plsc.subcore_barrier()

        def step(j, carry):
            pltpu.sync_copy(ones_v, cnt_sh.at[dst_v.at[j]], add=True)
            return carry

        lax.fori_loop(0, nchunk, step, 0, unroll=False)
        plsc.subcore_barrier()
        pltpu.sync_copy(cnt_sh.at[pl.ds(r0, rpt)],
                        out_hbm.at[c, pl.ds(r0, rpt)])

    return sc_cnt


def kernel(x, edge_index, Wl0, bl0, Wr0, Wl1, bl1, Wr1, Wl2, bl2, Wr2):
    n, d = x.shape
    e = edge_index.shape[1]

    ew = -(-e // (NW * CH)) * CH
    ep = ew * NW
    nchunk = ew // CH
    np_rows = -(-(n + LANES) // 1024) * 1024
    blk = 1024
    grid = (-(-n // blk),)

    src = edge_index[0]
    dst = edge_index[1]
    pad = ep - e
    src_p = jnp.concatenate([src, jnp.zeros((pad,), jnp.int32)]).reshape(NW, nchunk, CH)
    dst_p = jnp.concatenate([dst, jnp.full((pad,), n, jnp.int32)]).reshape(NW, nchunk, CH)
    zeros_d = jnp.zeros((np_rows, d), jnp.float32)
    ones_c = jnp.ones((CH, d), jnp.float32)

    sc_agg = _make_sc_agg(n, np_rows, d, nchunk)
    sc_cnt = _make_sc_cnt(np_rows, d, nchunk)

    w_spec = pl.BlockSpec((d, d), lambda i: (0, 0))
    b_spec = pl.BlockSpec((1, d), lambda i: (0, 0))
    h_spec = pl.BlockSpec((blk, d), lambda i: (i, 0))
    p_spec = pl.BlockSpec((NC, blk, d), lambda i: (0, i, 0))
    c_spec = p_spec
    h_sds = jax.ShapeDtypeStruct((n, d), jnp.float32)

    tc_pre = pl.pallas_call(
        _tc_pre_body, grid=grid,
        in_specs=[h_spec, w_spec, w_spec, b_spec],
        out_specs=[h_spec, h_spec],
        out_shape=[h_sds, h_sds],
    )
    tc_mid = pl.pallas_call(
        _tc_mid_body, grid=grid,
        in_specs=[p_spec, c_spec, h_spec, w_spec, w_spec, b_spec],
        out_specs=[h_spec, h_spec],
        out_shape=[h_sds, h_sds],
    )
    tc_fin = pl.pallas_call(
        _tc_fin_body, grid=grid,
        in_specs=[p_spec, c_spec, h_spec],
        out_specs=h_spec,
        out_shape=h_sds,
    )

    cnt = sc_cnt(dst_p, ones_c, zeros_d)

    m, r = tc_pre(x, Wl0, Wr0, bl0.reshape(1, d))
    p = sc_agg(m, src_p, dst_p, zeros_d)
    m, r = tc_mid(p, cnt, r, Wl1, Wr1, bl1.reshape(1, d))
    p = sc_agg(m, src_p, dst_p, zeros_d)
    m, r = tc_mid(p, cnt, r, Wl2, Wr2, bl2.reshape(1, d))
    p = sc_agg(m, src_p, dst_p, zeros_d)
    return tc_fin(p, cnt, r)
